# baseline (speedup 1.0000x reference)
import functools

import jax
import jax.numpy as jnp
from jax.experimental import pallas as pl
from jax.experimental.pallas import tpu as pltpu
from jax.experimental.pallas import tpu_sc as plsc

N_PROP = 5000
IMG = 800.0
S = 7
NUM_CLASSES = 21
SCORE_THRESH = 0.05
NMS_THRESH = 0.5
NUM_DET = 100
MIN_SIZE = 1.0

NCM1 = NUM_CLASSES - 1
ROWS = 32
LANES = 5120
NEG = -3.0e38

_D = 256
_B = N_PROP * S * S
_NW = 32
_CH = 56
_CHUNKS = 140
_NBUF = 4
_PER_W = _CH * _CHUNKS
_B_PAD = _NW * _PER_W
_N_PAD = _B_PAD // (S * S)
_QROWS = 99 * 99
_RB = 4480

assert _CHUNKS % _NBUF == 0 and _B_PAD == 5120 * 49


def _sc_gather(table, idx):
    mesh = plsc.VectorSubcoreMesh(core_axis_name="c", subcore_axis_name="s")

    @functools.partial(
        pl.kernel,
        out_type=jax.ShapeDtypeStruct((_B_PAD, _D), jnp.float32),
        mesh=mesh,
        scratch_types=[pltpu.VMEM((_PER_W,), jnp.int32)]
        + [pltpu.VMEM((_CH, _D), jnp.float32)] * _NBUF
        + [pltpu.SemaphoreType.DMA] * (2 * _NBUF),
    )
    def k(table_hbm, idx_hbm, out_hbm, idx_all, *bufs_and_sems):
        rows = bufs_and_sems[:_NBUF]
        sg = bufs_and_sems[_NBUF:2 * _NBUF]
        so = bufs_and_sems[2 * _NBUF:]
        wid = jax.lax.axis_index("s") * 2 + jax.lax.axis_index("c")
        base_w = wid * _PER_W
        pltpu.sync_copy(idx_hbm.at[pl.ds(base_w, _PER_W)], idx_all)

        def g_start(c, j):
            pltpu.async_copy(
                table_hbm.at[idx_all.at[pl.ds(c * _CH, _CH)]], rows[j], sg[j])

        def g_wait(c, j):
            pltpu.make_async_copy(
                table_hbm.at[idx_all.at[pl.ds(c * _CH, _CH)]], rows[j],
                sg[j]).wait()

        def o_start(c, j):
            pltpu.async_copy(
                rows[j], out_hbm.at[pl.ds(base_w + c * _CH, _CH)], so[j])

        def o_wait(c, j):
            pltpu.make_async_copy(
                rows[j], out_hbm.at[pl.ds(base_w + c * _CH, _CH)],
                so[j]).wait()

        for j in range(_NBUF):
            g_start(j, j)

        @pl.loop(0, _CHUNKS // _NBUF - 1)
        def _(g):
            c0 = g * _NBUF
            for j in range(_NBUF):
                g_wait(c0 + j, j)
                o_start(c0 + j, j)
            for j in range(_NBUF):
                o_wait(c0 + j, j)
                g_start(c0 + _NBUF + j, j)

        c0 = _CHUNKS - _NBUF
        for j in range(_NBUF):
            g_wait(c0 + j, j)
            o_start(c0 + j, j)
        for j in range(_NBUF):
            o_wait(c0 + j, j)

    return k(table, idx)


def _combine_kernel(q_ref, w_ref, s_ref):
    s_ref[...] = (q_ref[:, 0:64] * w_ref[:, 0:1]
                  + q_ref[:, 64:128] * w_ref[:, 1:2]
                  + q_ref[:, 128:192] * w_ref[:, 2:3]
                  + q_ref[:, 192:256] * w_ref[:, 3:4])


def _combine(q, w4):
    return pl.pallas_call(
        _combine_kernel,
        grid=(_B_PAD // _RB,),
        in_specs=[
            pl.BlockSpec((_RB, _D), lambda i: (i, 0)),
            pl.BlockSpec((_RB, 4), lambda i: (i, 0)),
        ],
        out_specs=pl.BlockSpec((_RB, 64), lambda i: (i, 0)),
        out_shape=jax.ShapeDtypeStruct((_B_PAD, 64), jnp.float32),
    )(q, w4)


def _nms_kernel(sc_ref, bx_ref, out_s_ref, out_b_ref):
    sc = sc_ref[...]
    x1 = bx_ref[0:1, :]
    y1 = bx_ref[1:2, :]
    x2 = bx_ref[2:3, :]
    y2 = bx_ref[3:4, :]
    a2 = (x2 - x1) * (y2 - y1)
    lane = jax.lax.broadcasted_iota(jnp.int32, (ROWS, LANES), 1)
    out_lane = jax.lax.broadcasted_iota(jnp.int32, (ROWS, NUM_DET), 1)

    out_s_ref[...] = jnp.zeros((ROWS, NUM_DET), jnp.float32)
    for c in range(4):
        out_b_ref[c, :, :] = jnp.zeros((ROWS, NUM_DET), jnp.float32)

    def body(t, sc):
        m = jnp.max(sc, axis=1, keepdims=True)
        idx = jnp.min(jnp.where(sc == m, lane, LANES), axis=1, keepdims=True)
        onehot = lane == idx
        zero = jnp.zeros((ROWS, LANES), jnp.float32)
        sx1 = jnp.sum(jnp.where(onehot, x1, zero), axis=1, keepdims=True)
        sy1 = jnp.sum(jnp.where(onehot, y1, zero), axis=1, keepdims=True)
        sx2 = jnp.sum(jnp.where(onehot, x2, zero), axis=1, keepdims=True)
        sy2 = jnp.sum(jnp.where(onehot, y2, zero), axis=1, keepdims=True)
        xx1 = jnp.maximum(sx1, x1)
        yy1 = jnp.maximum(sy1, y1)
        xx2 = jnp.minimum(sx2, x2)
        yy2 = jnp.minimum(sy2, y2)
        inter = jnp.maximum(xx2 - xx1, 0.0) * jnp.maximum(yy2 - yy1, 0.0)
        a1 = (sx2 - sx1) * (sy2 - sy1)
        iou = inter / (a1 + a2 - inter + 1e-9)
        sc = jnp.where(iou > NMS_THRESH, -1e9, sc)
        sc = jnp.where(onehot, -1e9, sc)
        hot_t = out_lane == t
        out_s_ref[...] = jnp.where(hot_t, m, out_s_ref[...])
        out_b_ref[0, :, :] = jnp.where(hot_t, sx1, out_b_ref[0, :, :])
        out_b_ref[1, :, :] = jnp.where(hot_t, sy1, out_b_ref[1, :, :])
        out_b_ref[2, :, :] = jnp.where(hot_t, sx2, out_b_ref[2, :, :])
        out_b_ref[3, :, :] = jnp.where(hot_t, sy2, out_b_ref[3, :, :])
        return sc

    jax.lax.fori_loop(0, NUM_DET, body, sc)


def _run_nms(sc0, bx):
    sc_pad = jnp.full((ROWS, LANES), NEG, jnp.float32)
    sc_pad = sc_pad.at[:NCM1, :N_PROP].set(sc0)
    bx_pad = jnp.zeros((4, LANES), jnp.float32)
    bx_pad = bx_pad.at[:, :N_PROP].set(bx.T)
    out_s, out_b = pl.pallas_call(
        _nms_kernel,
        out_shape=(
            jax.ShapeDtypeStruct((ROWS, NUM_DET), jnp.float32),
            jax.ShapeDtypeStruct((4, ROWS, NUM_DET), jnp.float32),
        ),
    )(sc_pad, bx_pad)
    boxes_k = jnp.transpose(out_b[:, :NCM1, :], (1, 2, 0))
    scores_k = out_s[:NCM1, :]
    return boxes_k, scores_k


def _axis_weights(coord, size):
    f = jnp.floor(coord)
    frac = coord - f
    fi = f.astype(jnp.int32)
    a = jnp.clip(fi, 0, size - 1)
    b = jnp.clip(fi + 1, 0, size - 1)
    p = jnp.clip(fi, 0, size - 2)
    one = jnp.float32(1.0)
    w_lo = (one - frac) * (a == p) + frac * (b == p)
    w_hi = (one - frac) * (a == p + 1) + frac * (b == p + 1)
    return p, w_lo, w_hi


def kernel(feature, proposal, image_shape, W_fc, b_fc, W_cls, b_cls, W_reg, b_reg):
    spatial_scale = feature.shape[-1] / IMG
    feat = feature[0]
    C, H, W = feat.shape
    b = proposal * spatial_scale
    px1, py1, px2, py2 = b[:, 0], b[:, 1], b[:, 2], b[:, 3]
    bw = jnp.maximum(px2 - px1, 1e-6)
    bh = jnp.maximum(py2 - py1, 1e-6)
    grid = (jnp.arange(S, dtype=jnp.float32) + 0.5) / S
    sx = px1[:, None] + grid[None, :] * bw[:, None]
    sy = py1[:, None] + grid[None, :] * bh[:, None]
    N = proposal.shape[0]
    yy = jnp.broadcast_to(sy[:, :, None], (N, S, S)) - 0.5
    xx = jnp.broadcast_to(sx[:, None, :], (N, S, S)) - 0.5

    pyi, wy_lo, wy_hi = _axis_weights(yy, H)
    pxi, wx_lo, wx_hi = _axis_weights(xx, W)
    qidx = (pyi * (W - 1) + pxi).reshape(-1)
    pad = (jnp.arange(_B_PAD - _B, dtype=jnp.int32) * 37) % _QROWS
    idx_full = jnp.concatenate([qidx, pad])

    T = jnp.transpose(feat, (1, 2, 0))
    quad = jnp.concatenate(
        [T[:-1, :-1], T[:-1, 1:], T[1:, :-1], T[1:, 1:]], axis=-1
    ).reshape(_QROWS, _D)

    q = _sc_gather(quad, idx_full)
    w00 = wy_lo * wx_lo
    w01 = wy_lo * wx_hi
    w10 = wy_hi * wx_lo
    w11 = wy_hi * wx_hi
    w4 = jnp.stack([w00, w01, w10, w11], axis=-1).reshape(_B, 4)
    w4 = jnp.concatenate(
        [w4, jnp.zeros((_B_PAD - _B, 4), jnp.float32)], axis=0)
    s_pc = _combine(q, w4).reshape(_N_PAD, S * S, C)
    bf = jnp.transpose(s_pc, (0, 2, 1)).reshape(_N_PAD, C * S * S)

    h = jax.nn.relu(bf @ W_fc + b_fc)[:N]
    class_logit = h @ W_cls + b_cls
    pred_score = jax.nn.softmax(class_logit, axis=-1)
    bx = jnp.stack([jnp.clip(proposal[:, 0], 0.0, IMG),
                    jnp.clip(proposal[:, 1], 0.0, IMG),
                    jnp.clip(proposal[:, 2], 0.0, IMG),
                    jnp.clip(proposal[:, 3], 0.0, IMG)], axis=1)
    w = bx[:, 2] - bx[:, 0]
    hh = bx[:, 3] - bx[:, 1]
    valid = (w >= MIN_SIZE) & (hh >= MIN_SIZE)
    scT = pred_score[:, 1:].T
    sc0 = jnp.where((scT >= SCORE_THRESH) & valid[None, :], scT, -1.0)
    boxes_k, scores_k = _run_nms(sc0, bx)
    labels_k = jnp.broadcast_to(jnp.arange(1, NUM_CLASSES)[:, None],
                                scores_k.shape)
    return (boxes_k.reshape(-1, 4), scores_k.reshape(-1), labels_k.reshape(-1))


# speedup vs baseline: 1.1063x; 1.1063x over previous
import functools

import jax
import jax.numpy as jnp
from jax.experimental import pallas as pl
from jax.experimental.pallas import tpu as pltpu
from jax.experimental.pallas import tpu_sc as plsc

N_PROP = 5000
IMG = 800.0
S = 7
NUM_CLASSES = 21
SCORE_THRESH = 0.05
NMS_THRESH = 0.5
NUM_DET = 100
MIN_SIZE = 1.0

NCM1 = NUM_CLASSES - 1
ROWS = 32
LANES = 5120
NEG = -3.0e38

_D = 256
_B = N_PROP * S * S
_NW = 32
_CH = 56
_CHUNKS = 140
_NBUF = 4
_PER_W = _CH * _CHUNKS
_B_PAD = _NW * _PER_W
_N_PAD = _B_PAD // (S * S)
_QROWS = 99 * 99
_RB = 4480

assert _CHUNKS % _NBUF == 0 and _B_PAD == 5120 * 49


def _sc_gather(table, idx):
    mesh = plsc.VectorSubcoreMesh(core_axis_name="c", subcore_axis_name="s")

    @functools.partial(
        pl.kernel,
        out_type=jax.ShapeDtypeStruct((_B_PAD, _D), jnp.float32),
        mesh=mesh,
        scratch_types=[pltpu.VMEM((_PER_W,), jnp.int32)]
        + [pltpu.VMEM((_CH, _D), jnp.float32)] * _NBUF
        + [pltpu.SemaphoreType.DMA] * (2 * _NBUF),
    )
    def k(table_hbm, idx_hbm, out_hbm, idx_all, *bufs_and_sems):
        rows = bufs_and_sems[:_NBUF]
        sg = bufs_and_sems[_NBUF:2 * _NBUF]
        so = bufs_and_sems[2 * _NBUF:]
        wid = jax.lax.axis_index("s") * 2 + jax.lax.axis_index("c")
        base_w = wid * _PER_W
        pltpu.sync_copy(idx_hbm.at[pl.ds(base_w, _PER_W)], idx_all)

        def g_start(c, j):
            pltpu.async_copy(
                table_hbm.at[idx_all.at[pl.ds(c * _CH, _CH)]], rows[j], sg[j])

        def g_wait(c, j):
            pltpu.make_async_copy(
                table_hbm.at[idx_all.at[pl.ds(c * _CH, _CH)]], rows[j],
                sg[j]).wait()

        def o_start(c, j):
            pltpu.async_copy(
                rows[j], out_hbm.at[pl.ds(base_w + c * _CH, _CH)], so[j])

        def o_wait(c, j):
            pltpu.make_async_copy(
                rows[j], out_hbm.at[pl.ds(base_w + c * _CH, _CH)],
                so[j]).wait()

        for j in range(_NBUF):
            g_start(j, j)

        @pl.loop(0, _CHUNKS // _NBUF - 1)
        def _(g):
            c0 = g * _NBUF
            for j in range(_NBUF):
                g_wait(c0 + j, j)
                o_start(c0 + j, j)
            for j in range(_NBUF):
                o_wait(c0 + j, j)
                g_start(c0 + _NBUF + j, j)

        c0 = _CHUNKS - _NBUF
        for j in range(_NBUF):
            g_wait(c0 + j, j)
            o_start(c0 + j, j)
        for j in range(_NBUF):
            o_wait(c0 + j, j)

    return k(table, idx)


def _combine_kernel(q_ref, w_ref, s_ref):
    q0 = q_ref[:, 0:64]
    q1 = q_ref[:, 64:128]
    q2 = q_ref[:, 128:192]
    q3 = q_ref[:, 192:256]
    ty0 = w_ref[:, 4:5] > 0.5
    ty1 = w_ref[:, 5:6] > 0.5
    tx0 = w_ref[:, 6:7] > 0.5
    tx1 = w_ref[:, 7:8] > 0.5
    v00 = jnp.where(ty0, jnp.where(tx0, q3, q2), jnp.where(tx0, q1, q0))
    v01 = jnp.where(ty0, jnp.where(tx1, q3, q2), jnp.where(tx1, q1, q0))
    v10 = jnp.where(ty1, jnp.where(tx0, q3, q2), jnp.where(tx0, q1, q0))
    v11 = jnp.where(ty1, jnp.where(tx1, q3, q2), jnp.where(tx1, q1, q0))
    s_ref[...] = (v00 * w_ref[:, 0:1] + v01 * w_ref[:, 1:2]
                  + v10 * w_ref[:, 2:3] + v11 * w_ref[:, 3:4])


def _combine(q, w8):
    return pl.pallas_call(
        _combine_kernel,
        grid=(_B_PAD // _RB,),
        in_specs=[
            pl.BlockSpec((_RB, _D), lambda i: (i, 0)),
            pl.BlockSpec((_RB, 8), lambda i: (i, 0)),
        ],
        out_specs=pl.BlockSpec((_RB, 64), lambda i: (i, 0)),
        out_shape=jax.ShapeDtypeStruct((_B_PAD, 64), jnp.float32),
    )(q, w8)


def _nms_kernel(sc_ref, bx_ref, out_s_ref, out_b_ref):
    sc = sc_ref[...]
    x1 = bx_ref[0:1, :]
    y1 = bx_ref[1:2, :]
    x2 = bx_ref[2:3, :]
    y2 = bx_ref[3:4, :]
    a2 = (x2 - x1) * (y2 - y1)
    lane = jax.lax.broadcasted_iota(jnp.int32, (ROWS, LANES), 1)
    out_lane = jax.lax.broadcasted_iota(jnp.int32, (ROWS, NUM_DET), 1)

    out_s_ref[...] = jnp.zeros((ROWS, NUM_DET), jnp.float32)
    for c in range(4):
        out_b_ref[c, :, :] = jnp.zeros((ROWS, NUM_DET), jnp.float32)

    def body(t, sc):
        m = jnp.max(sc, axis=1, keepdims=True)
        idx = jnp.min(jnp.where(sc == m, lane, LANES), axis=1, keepdims=True)
        onehot = lane == idx
        zero = jnp.zeros((ROWS, LANES), jnp.float32)
        sx1 = jnp.sum(jnp.where(onehot, x1, zero), axis=1, keepdims=True)
        sy1 = jnp.sum(jnp.where(onehot, y1, zero), axis=1, keepdims=True)
        sx2 = jnp.sum(jnp.where(onehot, x2, zero), axis=1, keepdims=True)
        sy2 = jnp.sum(jnp.where(onehot, y2, zero), axis=1, keepdims=True)
        xx1 = jnp.maximum(sx1, x1)
        yy1 = jnp.maximum(sy1, y1)
        xx2 = jnp.minimum(sx2, x2)
        yy2 = jnp.minimum(sy2, y2)
        inter = jnp.maximum(xx2 - xx1, 0.0) * jnp.maximum(yy2 - yy1, 0.0)
        a1 = (sx2 - sx1) * (sy2 - sy1)
        iou = inter / (a1 + a2 - inter + 1e-9)
        sc = jnp.where(iou > NMS_THRESH, -1e9, sc)
        sc = jnp.where(onehot, -1e9, sc)
        hot_t = out_lane == t
        out_s_ref[...] = jnp.where(hot_t, m, out_s_ref[...])
        out_b_ref[0, :, :] = jnp.where(hot_t, sx1, out_b_ref[0, :, :])
        out_b_ref[1, :, :] = jnp.where(hot_t, sy1, out_b_ref[1, :, :])
        out_b_ref[2, :, :] = jnp.where(hot_t, sx2, out_b_ref[2, :, :])
        out_b_ref[3, :, :] = jnp.where(hot_t, sy2, out_b_ref[3, :, :])
        return sc

    jax.lax.fori_loop(0, NUM_DET, body, sc)


def _run_nms(sc0, bx):
    sc_pad = jnp.full((ROWS, LANES), NEG, jnp.float32)
    sc_pad = sc_pad.at[:NCM1, :N_PROP].set(sc0)
    bx_pad = jnp.zeros((4, LANES), jnp.float32)
    bx_pad = bx_pad.at[:, :N_PROP].set(bx.T)
    out_s, out_b = pl.pallas_call(
        _nms_kernel,
        out_shape=(
            jax.ShapeDtypeStruct((ROWS, NUM_DET), jnp.float32),
            jax.ShapeDtypeStruct((4, ROWS, NUM_DET), jnp.float32),
        ),
    )(sc_pad, bx_pad)
    boxes_k = jnp.transpose(out_b[:, :NCM1, :], (1, 2, 0))
    scores_k = out_s[:NCM1, :]
    return boxes_k, scores_k


def _axis_taps(coord, size):
    f = jnp.floor(coord)
    frac = coord - f
    fi = f.astype(jnp.int32)
    a = jnp.clip(fi, 0, size - 1)
    b = jnp.clip(fi + 1, 0, size - 1)
    p = jnp.clip(fi, 0, size - 2)
    t_lo = (a == p + 1).astype(jnp.float32)
    t_hi = (b == p + 1).astype(jnp.float32)
    return p, frac, t_lo, t_hi


def kernel(feature, proposal, image_shape, W_fc, b_fc, W_cls, b_cls, W_reg, b_reg):
    spatial_scale = feature.shape[-1] / IMG
    feat = feature[0]
    C, H, W = feat.shape
    b = proposal * spatial_scale
    px1, py1, px2, py2 = b[:, 0], b[:, 1], b[:, 2], b[:, 3]
    bw = jnp.maximum(px2 - px1, 1e-6)
    bh = jnp.maximum(py2 - py1, 1e-6)
    grid = (jnp.arange(S, dtype=jnp.float32) + 0.5) / S
    sx = px1[:, None] + grid[None, :] * bw[:, None]
    sy = py1[:, None] + grid[None, :] * bh[:, None]
    N = proposal.shape[0]
    yy = jnp.broadcast_to(sy[:, :, None], (N, S, S)) - 0.5
    xx = jnp.broadcast_to(sx[:, None, :], (N, S, S)) - 0.5

    pyi, ly, ty0, ty1 = _axis_taps(yy, H)
    pxi, lx, tx0, tx1 = _axis_taps(xx, W)
    qidx = (pyi * (W - 1) + pxi).reshape(-1)
    pad = (jnp.arange(_B_PAD - _B, dtype=jnp.int32) * 37) % _QROWS
    idx_full = jnp.concatenate([qidx, pad])

    T = jnp.transpose(feat, (1, 2, 0))
    quad = jnp.concatenate(
        [T[:-1, :-1], T[:-1, 1:], T[1:, :-1], T[1:, 1:]], axis=-1
    ).reshape(_QROWS, _D)

    q = _sc_gather(quad, idx_full)
    def col(a):
        flat = a.reshape(_B)
        return jnp.concatenate(
            [flat, jnp.zeros((_B_PAD - _B,), flat.dtype)])[:, None]

    w00 = col((1 - ly) * (1 - lx))
    w01 = col((1 - ly) * lx)
    w10 = col(ly * (1 - lx))
    w11 = col(ly * lx)
    my0 = col(ty0) > 0.5
    my1 = col(ty1) > 0.5
    mx0 = col(tx0) > 0.5
    mx1 = col(tx1) > 0.5
    q0, q1, q2, q3 = q[:, 0:64], q[:, 64:128], q[:, 128:192], q[:, 192:256]
    v00 = jnp.where(my0, jnp.where(mx0, q3, q2), jnp.where(mx0, q1, q0))
    v01 = jnp.where(my0, jnp.where(mx1, q3, q2), jnp.where(mx1, q1, q0))
    v10 = jnp.where(my1, jnp.where(mx0, q3, q2), jnp.where(mx0, q1, q0))
    v11 = jnp.where(my1, jnp.where(mx1, q3, q2), jnp.where(mx1, q1, q0))
    s = v00 * w00 + v01 * w01 + v10 * w10 + v11 * w11
    s_pc = s.reshape(_N_PAD, S * S, C)
    bf = jnp.transpose(s_pc, (0, 2, 1)).reshape(_N_PAD, C * S * S)

    h = jax.nn.relu(bf @ W_fc + b_fc)[:N]
    class_logit = h @ W_cls + b_cls
    pred_score = jax.nn.softmax(class_logit, axis=-1)
    bx = jnp.stack([jnp.clip(proposal[:, 0], 0.0, IMG),
                    jnp.clip(proposal[:, 1], 0.0, IMG),
                    jnp.clip(proposal[:, 2], 0.0, IMG),
                    jnp.clip(proposal[:, 3], 0.0, IMG)], axis=1)
    w = bx[:, 2] - bx[:, 0]
    hh = bx[:, 3] - bx[:, 1]
    valid = (w >= MIN_SIZE) & (hh >= MIN_SIZE)
    scT = pred_score[:, 1:].T
    sc0 = jnp.where((scT >= SCORE_THRESH) & valid[None, :], scT, -1.0)
    boxes_k, scores_k = _run_nms(sc0, bx)
    labels_k = jnp.broadcast_to(jnp.arange(1, NUM_CLASSES)[:, None],
                                scores_k.shape)
    return (boxes_k.reshape(-1, 4), scores_k.reshape(-1), labels_k.reshape(-1))
